# baseline (speedup 1.0000x reference)
import functools

import jax
import jax.numpy as jnp
from jax import lax
from jax.experimental import pallas as pl
from jax.experimental.pallas import tpu as pltpu
from jax.experimental.pallas import tpu_sc as plsc

_N = 50000
_E = 800000
_DIN = 3
_H = 32
_D = _H + _DIN
_DP = 40
_DPB = 48
_ITERS = 3

_NPAD = 51200
_EPAD = 819200
_CH = 128
_NW = 32
_PERW = _EPAD // _NW
_SECT = 512
_QS = _SECT // _CH
_NSECT = _PERW // _SECT
_CROW = _PERW // _CH
_NACC = 50048
_RT = _NACC // 16
_BN = 2048
_BE = 8192

_SC_PARAMS = pltpu.CompilerParams(use_tc_tiling_on_sc=False)


def _dot(a, b):
    return lax.dot_general(a, b, (((1,), (0,)), ((), ())),
                           preferred_element_type=jnp.float32)


def _padcols(m):
    return jnp.concatenate(
        [m, jnp.zeros((m.shape[0], _DP - _D), jnp.float32)], axis=1)


def _enc_body(x_ref, w_ref, b_ref, o_ref, o16_ref):
    x = x_ref[...]
    y = _dot(x, w_ref[...]) + b_ref[...]
    h = _padcols(jnp.concatenate([y, x], axis=1))
    o_ref[...] = h
    o16_ref[...] = jnp.concatenate(
        [h, jnp.zeros((h.shape[0], _DPB - _DP), jnp.float32)],
        axis=1).astype(jnp.bfloat16)


def _enc_call(xp, We, be):
    return pl.pallas_call(
        _enc_body,
        grid=(_NPAD // _BN,),
        in_specs=[pl.BlockSpec((_BN, _DIN), lambda i: (i, 0)),
                  pl.BlockSpec((_DIN, _H), lambda i: (0, 0)),
                  pl.BlockSpec((1, _H), lambda i: (0, 0))],
        out_specs=[pl.BlockSpec((_BN, _DP), lambda i: (i, 0)),
                   pl.BlockSpec((_BN, _DPB), lambda i: (i, 0))],
        out_shape=[jax.ShapeDtypeStruct((_NPAD, _DP), jnp.float32),
                   jax.ShapeDtypeStruct((_NPAD, _DPB), jnp.bfloat16)],
    )(xp, We, be)


def _edge_body(hst_ref, w1_ref, b1_ref, w2_ref, b2_ref, w3_ref, b3_ref,
               mi_ref, mo_ref):
    hst = hst_ref[...]
    z = jnp.maximum(_dot(hst, w1_ref[...]) + b1_ref[...], 0.0)
    z = jnp.maximum(_dot(z.astype(jnp.bfloat16), w2_ref[...]) + b2_ref[...],
                    0.0)
    e = jax.nn.sigmoid(jnp.maximum(
        _dot(z.astype(jnp.bfloat16), w3_ref[...]) + b3_ref[...], 0.0))
    rows = lax.broadcasted_iota(jnp.int32, (_BE, 1), 0) + pl.program_id(0) * _BE
    e = jnp.where(rows < _E, e, 0.0)
    mi_ref[...] = hst[:, _DPB:_DPB + _DP].astype(jnp.float32) * e
    mo_ref[...] = hst[:, :_DP].astype(jnp.float32) * e


def _edge_call(hst, w1, b1, w2, b2, w3, b3):
    wspec = [pl.BlockSpec((2 * _DPB, _H), lambda i: (0, 0)),
             pl.BlockSpec((1, _H), lambda i: (0, 0)),
             pl.BlockSpec((_H, _H), lambda i: (0, 0)),
             pl.BlockSpec((1, _H), lambda i: (0, 0)),
             pl.BlockSpec((_H, 1), lambda i: (0, 0)),
             pl.BlockSpec((1, 1), lambda i: (0, 0))]
    return pl.pallas_call(
        _edge_body,
        grid=(_EPAD // _BE,),
        in_specs=[pl.BlockSpec((_BE, 2 * _DPB), lambda i: (i, 0))] + wspec,
        out_specs=[pl.BlockSpec((_BE, _DP), lambda i: (i, 0)),
                   pl.BlockSpec((_BE, _DP), lambda i: (i, 0))],
        out_shape=[jax.ShapeDtypeStruct((_EPAD, _DP), jnp.float32),
                   jax.ShapeDtypeStruct((_EPAD, _DP), jnp.float32)],
        compiler_params=pltpu.CompilerParams(
            dimension_semantics=("arbitrary",)),
    )(hst, w1, b1, w2, b2, w3, b3)


def _node_body(last, h_ref, p0_ref, p1_ref, x_ref, w1h_ref, w1m_ref, b1_ref,
               w2_ref, b2_ref, w3_ref, b3_ref, o_ref, extra_ref=None):
    h = h_ref[...]
    msg = p0_ref[...] + p1_ref[...]
    z = jnp.maximum(_dot(h, w1h_ref[...]) + _dot(msg, w1m_ref[...])
                    + b1_ref[...], 0.0)
    z = jnp.maximum(_dot(z, w2_ref[...]) + b2_ref[...], 0.0)
    o = _dot(z, w3_ref[...]) + b3_ref[...]
    if last:
        o_ref[...] = jax.nn.sigmoid(o)
    else:
        hn = _padcols(jnp.concatenate(
            [jnp.maximum(o, 0.0), x_ref[...]], axis=1)) + h
        o_ref[...] = hn
        extra_ref[...] = jnp.concatenate(
            [hn, jnp.zeros((hn.shape[0], _DPB - _DP), jnp.float32)],
            axis=1).astype(jnp.bfloat16)


def _node_call(last, h, parts, xp, w1h, w1m, b1, w2, b2, w3, b3):
    fo = 1 if last else _H
    wspec = [pl.BlockSpec((_DP, _H), lambda i: (0, 0)),
             pl.BlockSpec((_DP, _H), lambda i: (0, 0)),
             pl.BlockSpec((1, _H), lambda i: (0, 0)),
             pl.BlockSpec((_H, _H), lambda i: (0, 0)),
             pl.BlockSpec((1, _H), lambda i: (0, 0)),
             pl.BlockSpec((_H, fo), lambda i: (0, 0)),
             pl.BlockSpec((1, fo), lambda i: (0, 0))]
    nblk = _NPAD // _BN
    return pl.pallas_call(
        functools.partial(_node_body, last),
        grid=(nblk,),
        in_specs=[pl.BlockSpec((_BN, _DP), lambda i: (i, 0)),
                  pl.BlockSpec((_BN, _DP), lambda i: (i, 0)),
                  pl.BlockSpec((_BN, _DP), lambda i: (i + nblk, 0)),
                  pl.BlockSpec((_BN, _DIN), lambda i: (i, 0))] + wspec,
        out_specs=[pl.BlockSpec((_BN, 1 if last else _DP), lambda i: (i, 0))]
        + ([] if last else [pl.BlockSpec((_BN, _DPB), lambda i: (i, 0))]),
        out_shape=[jax.ShapeDtypeStruct((_NPAD, 1 if last else _DP),
                                        jnp.float32)]
        + ([] if last
           else [jax.ShapeDtypeStruct((_NPAD, _DPB), jnp.bfloat16)]),
    )(h, parts, parts, xp, w1h, w1m, b1, w2, b2, w3, b3)


def _sc_mesh():
    return plsc.VectorSubcoreMesh(core_axis_name="c", subcore_axis_name="s",
                                  num_cores=2, num_subcores=16)


def _gather_body(h_hbm, s_hbm, t_hbm, o_hbm, idx_v, rows0, rows1, gsem):
    wid = lax.axis_index("s") * 2 + lax.axis_index("c")
    rbase = wid * _CROW
    ebase = wid * _PERW

    def run_dir(idx_hbm, col):
        pltpu.sync_copy(idx_hbm.at[pl.ds(rbase, _CROW)], idx_v)

        def fire(si, buf):
            for q in range(_QS):
                pltpu.async_copy(
                    h_hbm.at[idx_v.at[si * _QS + q]],
                    buf.at[pl.ds(q * _CH, _CH)], gsem)

        def drain(buf):
            for q in range(_QS):
                pltpu.make_async_copy(
                    h_hbm.at[pl.ds(0, _CH)],
                    buf.at[pl.ds(q * _CH, _CH)], gsem).wait()

        def store(si, buf):
            pltpu.sync_copy(
                buf, o_hbm.at[pl.ds(ebase + si * _SECT, _SECT),
                              pl.ds(col, _DPB)])

        fire(0, rows0)

        @pl.loop(0, _NSECT - 2, step=2)
        def _(si):
            drain(rows0)
            fire(si + 1, rows1)
            store(si, rows0)
            drain(rows1)
            fire(si + 2, rows0)
            store(si + 1, rows1)

        drain(rows0)
        fire(_NSECT - 1, rows1)
        store(_NSECT - 2, rows0)
        drain(rows1)
        store(_NSECT - 1, rows1)

    run_dir(s_hbm, 0)
    run_dir(t_hbm, _DPB)


def _scatter_body(mi_hbm, mo_hbm, s_hbm, t_hbm, z_hbm, out_hbm, idx_v, upd_v,
                  acc):
    c = lax.axis_index("c")
    s = lax.axis_index("s")
    pltpu.sync_copy(z_hbm.at[pl.ds(s * _RT, _RT)], acc.at[pl.ds(s * _RT, _RT)])
    plsc.subcore_barrier()
    rbase = (c * (_EPAD // 2) + s * _PERW) // _CH

    @pl.loop(0, _CROW)
    def _(ci):
        erow = (rbase + ci) * _CH
        pltpu.sync_copy(s_hbm.at[rbase + ci], idx_v)
        pltpu.sync_copy(mi_hbm.at[pl.ds(erow, _CH)], upd_v)
        pltpu.sync_copy(upd_v, acc.at[idx_v], add=True)
        pltpu.sync_copy(t_hbm.at[rbase + ci], idx_v)
        pltpu.sync_copy(mo_hbm.at[pl.ds(erow, _CH)], upd_v)
        pltpu.sync_copy(upd_v, acc.at[idx_v], add=True)

    plsc.subcore_barrier()
    pltpu.sync_copy(acc.at[pl.ds(s * _RT, _RT)],
                    out_hbm.at[pl.ds(c * _NPAD + s * _RT, _RT)])


def _gather_sc(h16, sp2, tp2):
    return pl.kernel(
        _gather_body,
        out_type=jax.ShapeDtypeStruct((_EPAD, 2 * _DPB), jnp.bfloat16),
        mesh=_sc_mesh(),
        compiler_params=_SC_PARAMS,
        scratch_types=[pltpu.VMEM((_CROW, _CH), jnp.int32),
                       pltpu.VMEM((_SECT, _DPB), jnp.bfloat16),
                       pltpu.VMEM((_SECT, _DPB), jnp.bfloat16),
                       pltpu.SemaphoreType.DMA],
    )(h16, sp2, tp2)


def _scatter_sc(m_in, m_out, sp2, tp2, zeros):
    return pl.kernel(
        _scatter_body,
        out_type=jax.ShapeDtypeStruct((2 * _NPAD, _DP), jnp.float32),
        mesh=_sc_mesh(),
        compiler_params=_SC_PARAMS,
        scratch_types=[pltpu.VMEM((_CH,), jnp.int32),
                       pltpu.VMEM((_CH, _DP), jnp.float32),
                       pltpu.VMEM_SHARED((_NACC, _DP), jnp.float32)],
    )(m_in, m_out, sp2, tp2, zeros)


def _prep_mlp(p, split):
    (W1, b1), (W2, b2), (W3, b3) = p
    pad = jnp.zeros((_DP - _D, _H), jnp.float32)
    w1a = jnp.concatenate([W1[:_D], pad], axis=0)
    w1b = jnp.concatenate([W1[_D:], pad], axis=0)
    if split:
        return (w1a, w1b, b1.reshape(1, -1), W2, b2.reshape(1, -1),
                W3, b3.reshape(1, -1))
    padb = jnp.zeros((_DPB - _DP, _H), jnp.float32)
    w1 = jnp.concatenate([w1a, padb, w1b, padb],
                         axis=0).astype(jnp.bfloat16)
    return (w1, b1.reshape(1, -1), W2.astype(jnp.bfloat16),
            b2.reshape(1, -1), W3.astype(jnp.bfloat16), b3.reshape(1, -1))


def kernel(x, edge_index, params):
    xp = jnp.pad(x, ((0, _NPAD - _N), (0, 0)))
    fill = jnp.arange(_EPAD - _E, dtype=jnp.int32)
    sp2 = jnp.concatenate([edge_index[0], fill]).reshape(_EPAD // _CH, _CH)
    tp2 = jnp.concatenate([edge_index[1], fill]).reshape(_EPAD // _CH, _CH)
    zeros = jnp.zeros((_NACC, _DP), jnp.float32)
    We, be = params["enc"]
    h, h16 = _enc_call(xp, We, be.reshape(1, -1))
    for i in range(_ITERS):
        hst = _gather_sc(h16, sp2, tp2)
        m_in, m_out = _edge_call(hst, *_prep_mlp(params["edge"][i], False))
        parts = _scatter_sc(m_in, m_out, sp2, tp2, zeros)
        if i == _ITERS - 1:
            (out,) = _node_call(True, h, parts, xp,
                                *_prep_mlp(params["out"], True))
            return out[:_N]
        h, h16 = _node_call(False, h, parts, xp,
                            *_prep_mlp(params["node"][i], True))


# speedup vs baseline: 1.2686x; 1.2686x over previous
import functools

import jax
import jax.numpy as jnp
from jax import lax
from jax.experimental import pallas as pl
from jax.experimental.pallas import tpu as pltpu
from jax.experimental.pallas import tpu_sc as plsc

_N = 50000
_E = 800000
_DIN = 3
_H = 32
_D = _H + _DIN
_DP = 40
_DPB = 48
_ITERS = 3

_NPAD = 51200
_EPAD = 819200
_CH = 128
_NW = 32
_PERW = _EPAD // _NW
_SECT = 512
_QS = _SECT // _CH
_NSECT = _PERW // _SECT
_CROW = _PERW // _CH
_NACC = 50048
_RT = _NACC // 16
_BN = 2048
_BE = 8192

_SC_PARAMS = pltpu.CompilerParams(use_tc_tiling_on_sc=False)


def _dot(a, b):
    return lax.dot_general(a, b, (((1,), (0,)), ((), ())),
                           preferred_element_type=jnp.float32)


def _padcols(m):
    return jnp.concatenate(
        [m, jnp.zeros((m.shape[0], _DP - _D), jnp.float32)], axis=1)


def _enc_body(x_ref, w_ref, b_ref, o_ref):
    x = x_ref[...]
    y = _dot(x, w_ref[...]) + b_ref[...]
    o_ref[...] = _padcols(jnp.concatenate([y, x], axis=1))


def _enc_call(xp, We, be):
    return pl.pallas_call(
        _enc_body,
        grid=(_NPAD // _BN,),
        in_specs=[pl.BlockSpec((_BN, _DIN), lambda i: (i, 0)),
                  pl.BlockSpec((_DIN, _H), lambda i: (0, 0)),
                  pl.BlockSpec((1, _H), lambda i: (0, 0))],
        out_specs=pl.BlockSpec((_BN, _DP), lambda i: (i, 0)),
        out_shape=jax.ShapeDtypeStruct((_NPAD, _DP), jnp.float32),
    )(xp, We, be)


def _edge_body(hst_ref, w1_ref, b1_ref, w2_ref, b2_ref, w3_ref, b3_ref,
               mi_ref, mo_ref):
    hst = hst_ref[:, : 2 * _DP]
    z = jnp.maximum(_dot(hst, w1_ref[...]) + b1_ref[...], 0.0)
    z = jnp.maximum(_dot(z, w2_ref[...]) + b2_ref[...], 0.0)
    e = jax.nn.sigmoid(jnp.maximum(_dot(z, w3_ref[...]) + b3_ref[...], 0.0))
    rows = lax.broadcasted_iota(jnp.int32, (_BE, 1), 0) + pl.program_id(0) * _BE
    e = jnp.where(rows < _E, e, 0.0)
    mi_ref[...] = hst[:, _DP:] * e
    mo_ref[...] = hst[:, :_DP] * e


def _edge_call(hst, w1, b1, w2, b2, w3, b3):
    wspec = [pl.BlockSpec((2 * _DP, _H), lambda i: (0, 0)),
             pl.BlockSpec((1, _H), lambda i: (0, 0)),
             pl.BlockSpec((_H, _H), lambda i: (0, 0)),
             pl.BlockSpec((1, _H), lambda i: (0, 0)),
             pl.BlockSpec((_H, 1), lambda i: (0, 0)),
             pl.BlockSpec((1, 1), lambda i: (0, 0))]
    return pl.pallas_call(
        _edge_body,
        grid=(_EPAD // _BE,),
        in_specs=[pl.BlockSpec((_BE, 128), lambda i: (i, 0))] + wspec,
        out_specs=[pl.BlockSpec((_BE, _DP), lambda i: (i, 0)),
                   pl.BlockSpec((_BE, _DP), lambda i: (i, 0))],
        out_shape=[jax.ShapeDtypeStruct((_EPAD, _DP), jnp.float32),
                   jax.ShapeDtypeStruct((_EPAD, _DP), jnp.float32)],
        compiler_params=pltpu.CompilerParams(
            dimension_semantics=("arbitrary",)),
    )(hst, w1, b1, w2, b2, w3, b3)


def _node_body(last, h_ref, p0_ref, p1_ref, x_ref, w1h_ref, w1m_ref, b1_ref,
               w2_ref, b2_ref, w3_ref, b3_ref, o_ref):
    h = h_ref[...]
    msg = p0_ref[...] + p1_ref[...]
    z = jnp.maximum(_dot(h, w1h_ref[...]) + _dot(msg, w1m_ref[...])
                    + b1_ref[...], 0.0)
    z = jnp.maximum(_dot(z, w2_ref[...]) + b2_ref[...], 0.0)
    o = _dot(z, w3_ref[...]) + b3_ref[...]
    if last:
        o_ref[...] = jax.nn.sigmoid(o)
    else:
        o_ref[...] = _padcols(jnp.concatenate(
            [jnp.maximum(o, 0.0), x_ref[...]], axis=1)) + h


def _node_call(last, h, parts, xp, w1h, w1m, b1, w2, b2, w3, b3):
    fo = 1 if last else _H
    wspec = [pl.BlockSpec((_DP, _H), lambda i: (0, 0)),
             pl.BlockSpec((_DP, _H), lambda i: (0, 0)),
             pl.BlockSpec((1, _H), lambda i: (0, 0)),
             pl.BlockSpec((_H, _H), lambda i: (0, 0)),
             pl.BlockSpec((1, _H), lambda i: (0, 0)),
             pl.BlockSpec((_H, fo), lambda i: (0, 0)),
             pl.BlockSpec((1, fo), lambda i: (0, 0))]
    nblk = _NPAD // _BN
    return pl.pallas_call(
        functools.partial(_node_body, last),
        grid=(nblk,),
        in_specs=[pl.BlockSpec((_BN, _DP), lambda i: (i, 0)),
                  pl.BlockSpec((_BN, _DP), lambda i: (i, 0)),
                  pl.BlockSpec((_BN, _DP), lambda i: (i + nblk, 0)),
                  pl.BlockSpec((_BN, _DIN), lambda i: (i, 0))] + wspec,
        out_specs=pl.BlockSpec((_BN, 1 if last else _DP), lambda i: (i, 0)),
        out_shape=jax.ShapeDtypeStruct((_NPAD, 1 if last else _DP),
                                       jnp.float32),
    )(h, parts, parts, xp, w1h, w1m, b1, w2, b2, w3, b3)


def _sc_mesh():
    return plsc.VectorSubcoreMesh(core_axis_name="c", subcore_axis_name="s",
                                  num_cores=2, num_subcores=16)


def _gather_body(h_hbm, s_hbm, t_hbm, o_hbm, idx_v, rows0, rows1, gsem):
    wid = lax.axis_index("s") * 2 + lax.axis_index("c")
    rbase = wid * _CROW
    ebase = wid * _PERW

    def run_dir(idx_hbm, col):
        pltpu.sync_copy(idx_hbm.at[pl.ds(rbase, _CROW)], idx_v)

        def fire(si, buf):
            for q in range(_QS):
                pltpu.async_copy(
                    h_hbm.at[idx_v.at[si * _QS + q]],
                    buf.at[pl.ds(q * _CH, _CH)], gsem)

        def drain(buf):
            for q in range(_QS):
                pltpu.make_async_copy(
                    h_hbm.at[pl.ds(0, _CH)],
                    buf.at[pl.ds(q * _CH, _CH)], gsem).wait()

        def store(si, buf):
            pltpu.sync_copy(
                buf, o_hbm.at[pl.ds(ebase + si * _SECT, _SECT),
                              pl.ds(col, _DP)])

        fire(0, rows0)

        @pl.loop(0, _NSECT - 2, step=2)
        def _(si):
            drain(rows0)
            fire(si + 1, rows1)
            store(si, rows0)
            drain(rows1)
            fire(si + 2, rows0)
            store(si + 1, rows1)

        drain(rows0)
        fire(_NSECT - 1, rows1)
        store(_NSECT - 2, rows0)
        drain(rows1)
        store(_NSECT - 1, rows1)

    run_dir(s_hbm, 0)
    run_dir(t_hbm, _DP)


def _scatter_body(mi_hbm, mo_hbm, s_hbm, t_hbm, z_hbm, out_hbm, idx_v, upd_v,
                  acc):
    c = lax.axis_index("c")
    s = lax.axis_index("s")
    pltpu.sync_copy(z_hbm.at[pl.ds(s * _RT, _RT)], acc.at[pl.ds(s * _RT, _RT)])
    plsc.subcore_barrier()
    rbase = (c * (_EPAD // 2) + s * _PERW) // _CH

    @pl.loop(0, _CROW)
    def _(ci):
        erow = (rbase + ci) * _CH
        pltpu.sync_copy(s_hbm.at[rbase + ci], idx_v)
        pltpu.sync_copy(mi_hbm.at[pl.ds(erow, _CH)], upd_v)
        pltpu.sync_copy(upd_v, acc.at[idx_v], add=True)
        pltpu.sync_copy(t_hbm.at[rbase + ci], idx_v)
        pltpu.sync_copy(mo_hbm.at[pl.ds(erow, _CH)], upd_v)
        pltpu.sync_copy(upd_v, acc.at[idx_v], add=True)

    plsc.subcore_barrier()
    pltpu.sync_copy(acc.at[pl.ds(s * _RT, _RT)],
                    out_hbm.at[pl.ds(c * _NPAD + s * _RT, _RT)])


def _gather_sc(h, sp2, tp2):
    return pl.kernel(
        _gather_body,
        out_type=jax.ShapeDtypeStruct((_EPAD, 128), jnp.float32),
        mesh=_sc_mesh(),
        compiler_params=_SC_PARAMS,
        scratch_types=[pltpu.VMEM((_CROW, _CH), jnp.int32),
                       pltpu.VMEM((_SECT, _DP), jnp.float32),
                       pltpu.VMEM((_SECT, _DP), jnp.float32),
                       pltpu.SemaphoreType.DMA],
    )(h, sp2, tp2)


def _scatter_sc(m_in, m_out, sp2, tp2, zeros):
    return pl.kernel(
        _scatter_body,
        out_type=jax.ShapeDtypeStruct((2 * _NPAD, _DP), jnp.float32),
        mesh=_sc_mesh(),
        compiler_params=_SC_PARAMS,
        scratch_types=[pltpu.VMEM((_CH,), jnp.int32),
                       pltpu.VMEM((_CH, _DP), jnp.float32),
                       pltpu.VMEM_SHARED((_NACC, _DP), jnp.float32)],
    )(m_in, m_out, sp2, tp2, zeros)


def _prep_mlp(p, split):
    (W1, b1), (W2, b2), (W3, b3) = p
    pad = jnp.zeros((_DP - _D, _H), jnp.float32)
    w1a = jnp.concatenate([W1[:_D], pad], axis=0)
    w1b = jnp.concatenate([W1[_D:], pad], axis=0)
    if split:
        return (w1a, w1b, b1.reshape(1, -1), W2, b2.reshape(1, -1),
                W3, b3.reshape(1, -1))
    w1 = jnp.concatenate([w1a, w1b], axis=0)
    return (w1, b1.reshape(1, -1), W2, b2.reshape(1, -1),
            W3, b3.reshape(1, -1))


def kernel(x, edge_index, params):
    xp = jnp.pad(x, ((0, _NPAD - _N), (0, 0)))
    fill = jnp.arange(_EPAD - _E, dtype=jnp.int32)
    sp2 = jnp.concatenate([edge_index[0], fill]).reshape(_EPAD // _CH, _CH)
    tp2 = jnp.concatenate([edge_index[1], fill]).reshape(_EPAD // _CH, _CH)
    zeros = jnp.zeros((_NACC, _DP), jnp.float32)
    We, be = params["enc"]
    h = _enc_call(xp, We, be.reshape(1, -1))
    for i in range(_ITERS):
        hst = _gather_sc(h, sp2, tp2)
        m_in, m_out = _edge_call(hst, *_prep_mlp(params["edge"][i], False))
        parts = _scatter_sc(m_in, m_out, sp2, tp2, zeros)
        if i == _ITERS - 1:
            out = _node_call(True, h, parts, xp,
                             *_prep_mlp(params["out"], True))
            return out[:_N]
        h = _node_call(False, h, parts, xp,
                       *_prep_mlp(params["node"][i], True))


# speedup vs baseline: 1.3034x; 1.0274x over previous
import functools

import jax
import jax.numpy as jnp
from jax import lax
from jax.experimental import pallas as pl
from jax.experimental.pallas import tpu as pltpu
from jax.experimental.pallas import tpu_sc as plsc

_N = 50000
_E = 800000
_DIN = 3
_H = 32
_D = _H + _DIN
_DP = 40
_DPB = 48
_ITERS = 3

_NPAD = 51200
_EPAD = 819200
_CH = 128
_NW = 32
_PERW = _EPAD // _NW
_SECT = 512
_QS = _SECT // _CH
_NSECT = _PERW // _SECT
_CROW = _PERW // _CH
_NACC = 50048
_RT = _NACC // 16
_BN = 2048
_BE = 8192

_SC_PARAMS = pltpu.CompilerParams(use_tc_tiling_on_sc=False)


def _dot(a, b):
    return lax.dot_general(a, b, (((1,), (0,)), ((), ())),
                           preferred_element_type=jnp.float32)


def _padcols(m):
    return jnp.concatenate(
        [m, jnp.zeros((m.shape[0], _DP - _D), jnp.float32)], axis=1)


def _enc_body(x_ref, w_ref, b_ref, o_ref):
    x = x_ref[...]
    y = _dot(x, w_ref[...]) + b_ref[...]
    o_ref[...] = _padcols(jnp.concatenate([y, x], axis=1))


def _enc_call(xp, We, be):
    return pl.pallas_call(
        _enc_body,
        grid=(_NPAD // _BN,),
        in_specs=[pl.BlockSpec((_BN, _DIN), lambda i: (i, 0)),
                  pl.BlockSpec((_DIN, _H), lambda i: (0, 0)),
                  pl.BlockSpec((1, _H), lambda i: (0, 0))],
        out_specs=pl.BlockSpec((_BN, _DP), lambda i: (i, 0)),
        out_shape=jax.ShapeDtypeStruct((_NPAD, _DP), jnp.float32),
    )(xp, We, be)


def _edge_body(hst_ref, w1_ref, b1_ref, w2_ref, b2_ref, w3_ref, b3_ref,
               mi_ref, mo_ref):
    hst = hst_ref[:, : 2 * _DP]
    z = jnp.maximum(_dot(hst, w1_ref[...]) + b1_ref[...], 0.0)
    z = jnp.maximum(_dot(z, w2_ref[...]) + b2_ref[...], 0.0)
    e = jax.nn.sigmoid(jnp.maximum(_dot(z, w3_ref[...]) + b3_ref[...], 0.0))
    rows = lax.broadcasted_iota(jnp.int32, (_BE, 1), 0) + pl.program_id(0) * _BE
    e = jnp.where(rows < _E, e, 0.0)
    mi_ref[...] = hst[:, _DP:] * e
    mo_ref[...] = hst[:, :_DP] * e


def _edge_call(hst, w1, b1, w2, b2, w3, b3):
    wspec = [pl.BlockSpec((2 * _DP, _H), lambda i: (0, 0)),
             pl.BlockSpec((1, _H), lambda i: (0, 0)),
             pl.BlockSpec((_H, _H), lambda i: (0, 0)),
             pl.BlockSpec((1, _H), lambda i: (0, 0)),
             pl.BlockSpec((_H, 1), lambda i: (0, 0)),
             pl.BlockSpec((1, 1), lambda i: (0, 0))]
    return pl.pallas_call(
        _edge_body,
        grid=(_EPAD // _BE,),
        in_specs=[pl.BlockSpec((_BE, 128), lambda i: (i, 0))] + wspec,
        out_specs=[pl.BlockSpec((_BE, _DP), lambda i: (i, 0)),
                   pl.BlockSpec((_BE, _DP), lambda i: (i, 0))],
        out_shape=[jax.ShapeDtypeStruct((_EPAD, _DP), jnp.float32),
                   jax.ShapeDtypeStruct((_EPAD, _DP), jnp.float32)],
        compiler_params=pltpu.CompilerParams(
            dimension_semantics=("arbitrary",)),
    )(hst, w1, b1, w2, b2, w3, b3)


def _node_body(last, h_ref, p0_ref, p1_ref, x_ref, w1h_ref, w1m_ref, b1_ref,
               w2_ref, b2_ref, w3_ref, b3_ref, o_ref):
    h = h_ref[...]
    msg = p0_ref[...] + p1_ref[...]
    z = jnp.maximum(_dot(h, w1h_ref[...]) + _dot(msg, w1m_ref[...])
                    + b1_ref[...], 0.0)
    z = jnp.maximum(_dot(z, w2_ref[...]) + b2_ref[...], 0.0)
    o = _dot(z, w3_ref[...]) + b3_ref[...]
    if last:
        o_ref[...] = jax.nn.sigmoid(o)
    else:
        o_ref[...] = _padcols(jnp.concatenate(
            [jnp.maximum(o, 0.0), x_ref[...]], axis=1)) + h


def _node_call(last, h, parts, xp, w1h, w1m, b1, w2, b2, w3, b3):
    fo = 1 if last else _H
    wspec = [pl.BlockSpec((_DP, _H), lambda i: (0, 0)),
             pl.BlockSpec((_DP, _H), lambda i: (0, 0)),
             pl.BlockSpec((1, _H), lambda i: (0, 0)),
             pl.BlockSpec((_H, _H), lambda i: (0, 0)),
             pl.BlockSpec((1, _H), lambda i: (0, 0)),
             pl.BlockSpec((_H, fo), lambda i: (0, 0)),
             pl.BlockSpec((1, fo), lambda i: (0, 0))]
    nblk = _NPAD // _BN
    return pl.pallas_call(
        functools.partial(_node_body, last),
        grid=(nblk,),
        in_specs=[pl.BlockSpec((_BN, _DP), lambda i: (i, 0)),
                  pl.BlockSpec((_BN, _DP), lambda i: (i, 0)),
                  pl.BlockSpec((_BN, _DP), lambda i: (i + nblk, 0)),
                  pl.BlockSpec((_BN, _DIN), lambda i: (i, 0))] + wspec,
        out_specs=pl.BlockSpec((_BN, 1 if last else _DP), lambda i: (i, 0)),
        out_shape=jax.ShapeDtypeStruct((_NPAD, 1 if last else _DP),
                                       jnp.float32),
    )(h, parts, parts, xp, w1h, w1m, b1, w2, b2, w3, b3)


def _sc_mesh():
    return plsc.VectorSubcoreMesh(core_axis_name="c", subcore_axis_name="s",
                                  num_cores=2, num_subcores=16)


def _gather_body(h_hbm, s_hbm, t_hbm, o_hbm, idx_v, rows0, rows1, gsem):
    wid = lax.axis_index("s") * 2 + lax.axis_index("c")
    rbase = wid * _CROW
    ebase = wid * _PERW

    def run_dir(idx_hbm, col):
        pltpu.sync_copy(idx_hbm.at[pl.ds(rbase, _CROW)], idx_v)

        def fire(si, buf):
            for q in range(_QS):
                pltpu.async_copy(
                    h_hbm.at[idx_v.at[si * _QS + q]],
                    buf.at[pl.ds(q * _CH, _CH)], gsem)

        def drain(buf):
            for q in range(_QS):
                pltpu.make_async_copy(
                    h_hbm.at[pl.ds(0, _CH)],
                    buf.at[pl.ds(q * _CH, _CH)], gsem).wait()

        def store(si, buf):
            pltpu.sync_copy(
                buf, o_hbm.at[pl.ds(ebase + si * _SECT, _SECT),
                              pl.ds(col, _DP)])

        fire(0, rows0)

        @pl.loop(0, _NSECT - 2, step=2)
        def _(si):
            drain(rows0)
            fire(si + 1, rows1)
            store(si, rows0)
            drain(rows1)
            fire(si + 2, rows0)
            store(si + 1, rows1)

        drain(rows0)
        fire(_NSECT - 1, rows1)
        store(_NSECT - 2, rows0)
        drain(rows1)
        store(_NSECT - 1, rows1)

    run_dir(s_hbm, 0)
    run_dir(t_hbm, _DP)


def _scatter_body(mi_hbm, mo_hbm, s_hbm, t_hbm, z_hbm, out_hbm, idx0, upd0,
                  idx1, upd1, lsem, acc):
    c = lax.axis_index("c")
    s = lax.axis_index("s")
    pltpu.sync_copy(z_hbm.at[pl.ds(s * _RT, _RT)], acc.at[pl.ds(s * _RT, _RT)])
    plsc.subcore_barrier()
    rbase = (c * (_EPAD // 2) + s * _PERW) // _CH

    hc = _CH // 2

    def srcs(k, ci):
        idx_hbm = s_hbm if k < 2 else t_hbm
        upd_hbm = mi_hbm if k < 2 else mo_hbm
        col = (k % 2) * hc
        return (idx_hbm.at[rbase + ci, pl.ds(col, hc)],
                upd_hbm.at[pl.ds((rbase + ci) * _CH + col, hc)])

    def fire(k, ci, ib, ub):
        i_src, u_src = srcs(k, ci)
        pltpu.async_copy(i_src, ib, lsem)
        pltpu.async_copy(u_src, ub, lsem)

    def drain(ib, ub):
        i_src, u_src = srcs(0, 0)
        pltpu.make_async_copy(i_src, ib, lsem).wait()
        pltpu.make_async_copy(u_src, ub, lsem).wait()

    bufs = [(idx0, upd0), (idx1, upd1)]
    fire(0, 0, *bufs[0])

    @pl.loop(0, _CROW)
    def _(ci):
        for k in range(4):
            ib, ub = bufs[k % 2]
            drain(ib, ub)
            if k < 3:
                fire(k + 1, ci, *bufs[(k + 1) % 2])
            else:
                fire(0, lax.min(ci + 1, _CROW - 1), *bufs[(k + 1) % 2])
            pltpu.sync_copy(ub, acc.at[ib], add=True)

    drain(*bufs[0])

    plsc.subcore_barrier()
    pltpu.sync_copy(acc.at[pl.ds(s * _RT, _RT)],
                    out_hbm.at[pl.ds(c * _NPAD + s * _RT, _RT)])


def _gather_sc(h, sp2, tp2):
    return pl.kernel(
        _gather_body,
        out_type=jax.ShapeDtypeStruct((_EPAD, 128), jnp.float32),
        mesh=_sc_mesh(),
        compiler_params=_SC_PARAMS,
        scratch_types=[pltpu.VMEM((_CROW, _CH), jnp.int32),
                       pltpu.VMEM((_SECT, _DP), jnp.float32),
                       pltpu.VMEM((_SECT, _DP), jnp.float32),
                       pltpu.SemaphoreType.DMA],
    )(h, sp2, tp2)


def _scatter_sc(m_in, m_out, sp2, tp2, zeros):
    return pl.kernel(
        _scatter_body,
        out_type=jax.ShapeDtypeStruct((2 * _NPAD, _DP), jnp.float32),
        mesh=_sc_mesh(),
        compiler_params=_SC_PARAMS,
        scratch_types=[pltpu.VMEM((_CH // 2,), jnp.int32),
                       pltpu.VMEM((_CH // 2, _DP), jnp.float32),
                       pltpu.VMEM((_CH // 2,), jnp.int32),
                       pltpu.VMEM((_CH // 2, _DP), jnp.float32),
                       pltpu.SemaphoreType.DMA,
                       pltpu.VMEM_SHARED((_NACC, _DP), jnp.float32)],
    )(m_in, m_out, sp2, tp2, zeros)


def _prep_mlp(p, split):
    (W1, b1), (W2, b2), (W3, b3) = p
    pad = jnp.zeros((_DP - _D, _H), jnp.float32)
    w1a = jnp.concatenate([W1[:_D], pad], axis=0)
    w1b = jnp.concatenate([W1[_D:], pad], axis=0)
    if split:
        return (w1a, w1b, b1.reshape(1, -1), W2, b2.reshape(1, -1),
                W3, b3.reshape(1, -1))
    w1 = jnp.concatenate([w1a, w1b], axis=0)
    return (w1, b1.reshape(1, -1), W2, b2.reshape(1, -1),
            W3, b3.reshape(1, -1))


def kernel(x, edge_index, params):
    xp = jnp.pad(x, ((0, _NPAD - _N), (0, 0)))
    fill = jnp.arange(_EPAD - _E, dtype=jnp.int32)
    sp2 = jnp.concatenate([edge_index[0], fill]).reshape(_EPAD // _CH, _CH)
    tp2 = jnp.concatenate([edge_index[1], fill]).reshape(_EPAD // _CH, _CH)
    zeros = jnp.zeros((_NACC, _DP), jnp.float32)
    We, be = params["enc"]
    h = _enc_call(xp, We, be.reshape(1, -1))
    for i in range(_ITERS):
        hst = _gather_sc(h, sp2, tp2)
        m_in, m_out = _edge_call(hst, *_prep_mlp(params["edge"][i], False))
        parts = _scatter_sc(m_in, m_out, sp2, tp2, zeros)
        if i == _ITERS - 1:
            out = _node_call(True, h, parts, xp,
                             *_prep_mlp(params["out"], True))
            return out[:_N]
        h = _node_call(False, h, parts, xp,
                       *_prep_mlp(params["node"][i], True))


# speedup vs baseline: 1.3044x; 1.0008x over previous
import functools

import jax
import jax.numpy as jnp
from jax import lax
from jax.experimental import pallas as pl
from jax.experimental.pallas import tpu as pltpu
from jax.experimental.pallas import tpu_sc as plsc

_N = 50000
_E = 800000
_DIN = 3
_H = 32
_D = _H + _DIN
_DP = 40
_ITERS = 3

_NPAD = 51200
_EPAD = 819200
_CH = 128
_NW = 32
_PERW = _EPAD // _NW
_SECT = 512
_QS = _SECT // _CH
_NSECT = _PERW // _SECT
_CROW = _PERW // _CH
_NACC = 50048
_RT = _NACC // 16
_BN = 2048
_BE = 8192

_SC_PARAMS = pltpu.CompilerParams(use_tc_tiling_on_sc=False)


def _dot(a, b):
    return lax.dot_general(a, b, (((1,), (0,)), ((), ())),
                           preferred_element_type=jnp.float32)


def _padcols(m):
    return jnp.concatenate(
        [m, jnp.zeros((m.shape[0], _DP - _D), jnp.float32)], axis=1)


def _enc_body(x_ref, w_ref, b_ref, o_ref):
    x = x_ref[...]
    y = _dot(x, w_ref[...]) + b_ref[...]
    o_ref[...] = _padcols(jnp.concatenate([y, x], axis=1))


def _enc_call(xp, We, be):
    return pl.pallas_call(
        _enc_body,
        grid=(_NPAD // _BN,),
        in_specs=[pl.BlockSpec((_BN, _DIN), lambda i: (i, 0)),
                  pl.BlockSpec((_DIN, _H), lambda i: (0, 0)),
                  pl.BlockSpec((1, _H), lambda i: (0, 0))],
        out_specs=pl.BlockSpec((_BN, _DP), lambda i: (i, 0)),
        out_shape=jax.ShapeDtypeStruct((_NPAD, _DP), jnp.float32),
    )(xp, We, be)


def _edge_body(hst_ref, w1_ref, b1_ref, w2_ref, b2_ref, w3_ref, b3_ref,
               mi_ref, mo_ref):
    hst = hst_ref[:, : 2 * _DP]
    z = jnp.maximum(_dot(hst, w1_ref[...]) + b1_ref[...], 0.0)
    z = jnp.maximum(_dot(z, w2_ref[...]) + b2_ref[...], 0.0)
    e = jax.nn.sigmoid(jnp.maximum(_dot(z, w3_ref[...]) + b3_ref[...], 0.0))
    rows = lax.broadcasted_iota(jnp.int32, (_BE, 1), 0) + pl.program_id(0) * _BE
    e = jnp.where(rows < _E, e, 0.0)
    mi_ref[...] = hst[:, _DP:] * e
    mo_ref[...] = hst[:, :_DP] * e


def _edge_call(hst, w1, b1, w2, b2, w3, b3):
    wspec = [pl.BlockSpec((2 * _DP, _H), lambda i: (0, 0)),
             pl.BlockSpec((1, _H), lambda i: (0, 0)),
             pl.BlockSpec((_H, _H), lambda i: (0, 0)),
             pl.BlockSpec((1, _H), lambda i: (0, 0)),
             pl.BlockSpec((_H, 1), lambda i: (0, 0)),
             pl.BlockSpec((1, 1), lambda i: (0, 0))]
    return pl.pallas_call(
        _edge_body,
        grid=(_EPAD // _BE,),
        in_specs=[pl.BlockSpec((_BE, 128), lambda i: (i, 0))] + wspec,
        out_specs=[pl.BlockSpec((_BE, _DP), lambda i: (i, 0)),
                   pl.BlockSpec((_BE, _DP), lambda i: (i, 0))],
        out_shape=[jax.ShapeDtypeStruct((_EPAD, _DP), jnp.float32),
                   jax.ShapeDtypeStruct((_EPAD, _DP), jnp.float32)],
        compiler_params=pltpu.CompilerParams(
            dimension_semantics=("arbitrary",)),
    )(hst, w1, b1, w2, b2, w3, b3)


def _node_body(last, h_ref, p0_ref, p1_ref, x_ref, w1h_ref, w1m_ref, b1_ref,
               w2_ref, b2_ref, w3_ref, b3_ref, o_ref):
    h = h_ref[...]
    msg = p0_ref[...] + p1_ref[...]
    z = jnp.maximum(_dot(h, w1h_ref[...]) + _dot(msg, w1m_ref[...])
                    + b1_ref[...], 0.0)
    z = jnp.maximum(_dot(z, w2_ref[...]) + b2_ref[...], 0.0)
    o = _dot(z, w3_ref[...]) + b3_ref[...]
    if last:
        o_ref[...] = jax.nn.sigmoid(o)
    else:
        o_ref[...] = _padcols(jnp.concatenate(
            [jnp.maximum(o, 0.0), x_ref[...]], axis=1)) + h


def _node_call(last, h, parts, xp, w1h, w1m, b1, w2, b2, w3, b3):
    fo = 1 if last else _H
    wspec = [pl.BlockSpec((_DP, _H), lambda i: (0, 0)),
             pl.BlockSpec((_DP, _H), lambda i: (0, 0)),
             pl.BlockSpec((1, _H), lambda i: (0, 0)),
             pl.BlockSpec((_H, _H), lambda i: (0, 0)),
             pl.BlockSpec((1, _H), lambda i: (0, 0)),
             pl.BlockSpec((_H, fo), lambda i: (0, 0)),
             pl.BlockSpec((1, fo), lambda i: (0, 0))]
    nblk = _NPAD // _BN
    return pl.pallas_call(
        functools.partial(_node_body, last),
        grid=(nblk,),
        in_specs=[pl.BlockSpec((_BN, _DP), lambda i: (i, 0)),
                  pl.BlockSpec((_BN, _DP), lambda i: (i, 0)),
                  pl.BlockSpec((_BN, _DP), lambda i: (i + nblk, 0)),
                  pl.BlockSpec((_BN, _DIN), lambda i: (i, 0))] + wspec,
        out_specs=pl.BlockSpec((_BN, 1 if last else _DP), lambda i: (i, 0)),
        out_shape=jax.ShapeDtypeStruct((_NPAD, 1 if last else _DP),
                                       jnp.float32),
    )(h, parts, parts, xp, w1h, w1m, b1, w2, b2, w3, b3)


def _sc_mesh():
    return plsc.VectorSubcoreMesh(core_axis_name="c", subcore_axis_name="s",
                                  num_cores=2, num_subcores=16)


def _gather_body(h_hbm, s_hbm, t_hbm, o_hbm, idx_v, rows0, rows1, gsem):
    wid = lax.axis_index("s") * 2 + lax.axis_index("c")
    rbase = wid * _CROW
    ebase = wid * _PERW

    def run_dir(idx_hbm, col):
        pltpu.sync_copy(idx_hbm.at[pl.ds(rbase, _CROW)], idx_v)

        def fire(si, buf):
            for q in range(_QS):
                pltpu.async_copy(
                    h_hbm.at[idx_v.at[si * _QS + q]],
                    buf.at[pl.ds(q * _CH, _CH)], gsem)

        def drain(buf):
            for q in range(_QS):
                pltpu.make_async_copy(
                    h_hbm.at[pl.ds(0, _CH)],
                    buf.at[pl.ds(q * _CH, _CH)], gsem).wait()

        def store(si, buf):
            pltpu.sync_copy(
                buf, o_hbm.at[pl.ds(ebase + si * _SECT, _SECT),
                              pl.ds(col, _DP)])

        fire(0, rows0)

        @pl.loop(0, _NSECT - 2, step=2)
        def _(si):
            drain(rows0)
            fire(si + 1, rows1)
            store(si, rows0)
            drain(rows1)
            fire(si + 2, rows0)
            store(si + 1, rows1)

        drain(rows0)
        fire(_NSECT - 1, rows1)
        store(_NSECT - 2, rows0)
        drain(rows1)
        store(_NSECT - 1, rows1)

    run_dir(s_hbm, 0)
    run_dir(t_hbm, _DP)


def _scatter_body(mi_hbm, mo_hbm, s_hbm, t_hbm, z_hbm, out_hbm, idx0, upd0,
                  idx1, upd1, lsem, acc):
    c = lax.axis_index("c")
    s = lax.axis_index("s")
    pltpu.sync_copy(z_hbm.at[pl.ds(s * _RT, _RT)], acc.at[pl.ds(s * _RT, _RT)])
    plsc.subcore_barrier()
    rbase = (c * (_EPAD // 2) + s * _PERW) // _CH

    hc = _CH // 2

    def srcs(k, ci):
        idx_hbm = s_hbm if k < 2 else t_hbm
        upd_hbm = mi_hbm if k < 2 else mo_hbm
        col = (k % 2) * hc
        return (idx_hbm.at[rbase + ci, pl.ds(col, hc)],
                upd_hbm.at[pl.ds((rbase + ci) * _CH + col, hc)])

    def fire(k, ci, ib, ub):
        i_src, u_src = srcs(k, ci)
        pltpu.async_copy(i_src, ib, lsem)
        pltpu.async_copy(u_src, ub, lsem)

    def drain(ib, ub):
        i_src, u_src = srcs(0, 0)
        pltpu.make_async_copy(i_src, ib, lsem).wait()
        pltpu.make_async_copy(u_src, ub, lsem).wait()

    bufs = [(idx0, upd0), (idx1, upd1)]
    fire(0, 0, *bufs[0])

    @pl.loop(0, _CROW)
    def _(ci):
        for k in range(4):
            ib, ub = bufs[k % 2]
            drain(ib, ub)
            if k < 3:
                fire(k + 1, ci, *bufs[(k + 1) % 2])
            else:
                fire(0, lax.min(ci + 1, _CROW - 1), *bufs[(k + 1) % 2])
            pltpu.sync_copy(ub, acc.at[ib], add=True)

    drain(*bufs[0])

    plsc.subcore_barrier()
    pltpu.sync_copy(acc.at[pl.ds(s * _RT, _RT)],
                    out_hbm.at[pl.ds(c * _NPAD + s * _RT, _RT)])


def _gather_sc(h, sp2, tp2):
    return pl.kernel(
        _gather_body,
        out_type=jax.ShapeDtypeStruct((_EPAD, 128), jnp.float32),
        mesh=_sc_mesh(),
        compiler_params=_SC_PARAMS,
        scratch_types=[pltpu.VMEM((_CROW, _CH), jnp.int32),
                       pltpu.VMEM((_SECT, _DP), jnp.float32),
                       pltpu.VMEM((_SECT, _DP), jnp.float32),
                       pltpu.SemaphoreType.DMA],
    )(h, sp2, tp2)


def _scatter_sc(m_in, m_out, sp2, tp2, zeros):
    return pl.kernel(
        _scatter_body,
        out_type=jax.ShapeDtypeStruct((2 * _NPAD, _DP), jnp.float32),
        mesh=_sc_mesh(),
        compiler_params=_SC_PARAMS,
        scratch_types=[pltpu.VMEM((_CH // 2,), jnp.int32),
                       pltpu.VMEM((_CH // 2, _DP), jnp.float32),
                       pltpu.VMEM((_CH // 2,), jnp.int32),
                       pltpu.VMEM((_CH // 2, _DP), jnp.float32),
                       pltpu.SemaphoreType.DMA,
                       pltpu.VMEM_SHARED((_NACC, _DP), jnp.float32)],
    )(m_in, m_out, sp2, tp2, zeros)


def _prep_mlp(p, split):
    (W1, b1), (W2, b2), (W3, b3) = p
    pad = jnp.zeros((_DP - _D, _H), jnp.float32)
    w1a = jnp.concatenate([W1[:_D], pad], axis=0)
    w1b = jnp.concatenate([W1[_D:], pad], axis=0)
    if split:
        return (w1a, w1b, b1.reshape(1, -1), W2, b2.reshape(1, -1),
                W3, b3.reshape(1, -1))
    w1 = jnp.concatenate([w1a, w1b], axis=0)
    return (w1, b1.reshape(1, -1), W2, b2.reshape(1, -1),
            W3, b3.reshape(1, -1))


def kernel(x, edge_index, params):
    xp = jnp.pad(x, ((0, _NPAD - _N), (0, 0)))
    fill = jnp.arange(_EPAD - _E, dtype=jnp.int32)
    sp2 = jnp.concatenate([edge_index[0], fill]).reshape(_EPAD // _CH, _CH)
    tp2 = jnp.concatenate([edge_index[1], fill]).reshape(_EPAD // _CH, _CH)
    zeros = jnp.zeros((_NACC, _DP), jnp.float32)
    We, be = params["enc"]
    h = _enc_call(xp, We, be.reshape(1, -1))
    for i in range(_ITERS):
        hst = _gather_sc(h, sp2, tp2)
        m_in, m_out = _edge_call(hst, *_prep_mlp(params["edge"][i], False))
        parts = _scatter_sc(m_in, m_out, sp2, tp2, zeros)
        if i == _ITERS - 1:
            out = _node_call(True, h, parts, xp,
                             *_prep_mlp(params["out"], True))
            return out[:_N]
        h = _node_call(False, h, parts, xp,
                       *_prep_mlp(params["node"][i], True))
